# baseline (speedup 1.0000x reference)
import functools

import jax
import jax.numpy as jnp
from jax import lax
from jax.experimental import pallas as pl
from jax.experimental.pallas import tpu as pltpu
from jax.experimental.pallas import tpu_sc as plsc

_NSEG = 10000
_NEDGE = 320000
_D = 128
_NC = 2
_NS = 16
_NW = _NC * _NS
_EDGES_PER_TILE = _NEDGE // _NW
_K = 128
_NITER = _EDGES_PER_TILE // _K
_KT = _EDGES_PER_TILE - _NITER * _K
_NBUF = 2
_RPT = 624
_TAIL = _NSEG - _RPT * _NS

_mesh = plsc.VectorSubcoreMesh(core_axis_name="c", subcore_axis_name="s")


@functools.partial(
    pl.kernel,
    out_type=(
        jax.ShapeDtypeStruct((_NSEG, _D), jnp.float32),
        jax.ShapeDtypeStruct((_NSEG, _D), jnp.float32),
    ),
    mesh=_mesh,
    scratch_types=[
        [pltpu.VMEM((_K,), jnp.int32) for _ in range(_NBUF)],
        [pltpu.VMEM((_K, _D), jnp.float32) for _ in range(_NBUF)],
        pltpu.VMEM((_KT,), jnp.int32),
        pltpu.VMEM((_KT, _D), jnp.float32),
        pltpu.VMEM_SHARED((_NSEG, _D), jnp.float32),
        [pltpu.SemaphoreType.DMA for _ in range(_NBUF)],
    ],
)
def _sc_partials(
    feat_hbm, ids_hbm, zeros_hbm, out0, out1, idbufs, bufs, idtail, ftail, acc, sems
):
    c = lax.axis_index("c")
    s = lax.axis_index("s")
    wid = s * _NC + c
    base = wid * _EDGES_PER_TILE
    r0 = s * _RPT

    for b in range(_NBUF):
        pltpu.make_async_copy(
            ids_hbm.at[pl.ds(base + b * _K, _K)], idbufs[b], sems[b]
        ).start()
        pltpu.make_async_copy(
            feat_hbm.at[pl.ds(base + b * _K, _K)], bufs[b], sems[b]
        ).start()
    pltpu.sync_copy(zeros_hbm.at[pl.ds(r0, _RPT)], acc.at[pl.ds(r0, _RPT)])

    @pl.when(s == 0)
    def _():
        pltpu.sync_copy(
            zeros_hbm.at[pl.ds(_RPT * _NS, _TAIL)], acc.at[pl.ds(_RPT * _NS, _TAIL)]
        )

    plsc.subcore_barrier()

    def run_block(g, idbuf, buf, sem):
        off = base + g * _K
        pltpu.make_async_copy(ids_hbm.at[pl.ds(off, _K)], idbuf, sem).wait()
        pltpu.make_async_copy(feat_hbm.at[pl.ds(off, _K)], buf, sem).wait()
        pltpu.sync_copy(buf, acc.at[idbuf], add=True)

        @pl.when(g + _NBUF < _NITER)
        def _():
            off2 = base + (g + _NBUF) * _K
            pltpu.make_async_copy(ids_hbm.at[pl.ds(off2, _K)], idbuf, sem).start()
            pltpu.make_async_copy(feat_hbm.at[pl.ds(off2, _K)], buf, sem).start()

    def body(g, carry):
        for b in range(_NBUF):

            @pl.when(g % _NBUF == b)
            def _(b=b):
                run_block(g, idbufs[b], bufs[b], sems[b])

        return carry

    lax.fori_loop(0, _NITER, body, 0)

    toff = base + _NITER * _K
    pltpu.sync_copy(ids_hbm.at[pl.ds(toff, _KT)], idtail)
    pltpu.sync_copy(feat_hbm.at[pl.ds(toff, _KT)], ftail)
    pltpu.sync_copy(ftail, acc.at[idtail], add=True)
    plsc.subcore_barrier()

    @pl.when(c == 0)
    def _():
        pltpu.sync_copy(acc.at[pl.ds(r0, _RPT)], out0.at[pl.ds(r0, _RPT)])

        @pl.when(s == 0)
        def _():
            pltpu.sync_copy(
                acc.at[pl.ds(_RPT * _NS, _TAIL)], out0.at[pl.ds(_RPT * _NS, _TAIL)]
            )

    @pl.when(c == 1)
    def _():
        pltpu.sync_copy(acc.at[pl.ds(r0, _RPT)], out1.at[pl.ds(r0, _RPT)])

        @pl.when(s == 0)
        def _():
            pltpu.sync_copy(
                acc.at[pl.ds(_RPT * _NS, _TAIL)], out1.at[pl.ds(_RPT * _NS, _TAIL)]
            )


def _add_body(a_ref, b_ref, o_ref):
    o_ref[...] = a_ref[...] + b_ref[...]


_combine = pl.pallas_call(
    _add_body,
    grid=(10,),
    in_specs=[
        pl.BlockSpec((_NSEG // 10, _D), lambda i: (i, 0)),
        pl.BlockSpec((_NSEG // 10, _D), lambda i: (i, 0)),
    ],
    out_specs=pl.BlockSpec((_NSEG // 10, _D), lambda i: (i, 0)),
    out_shape=jax.ShapeDtypeStruct((_NSEG, _D), jnp.float32),
)


def kernel(features, segment_ids):
    zeros = jnp.zeros((_NSEG, _D), jnp.float32)
    p0, p1 = _sc_partials(features, segment_ids, zeros)
    return _combine(p0, p1)


# speedup vs baseline: 1.0814x; 1.0814x over previous
import functools

import numpy as np

import jax
import jax.numpy as jnp
from jax import lax
from jax.experimental import pallas as pl
from jax.experimental.pallas import tpu as pltpu
from jax.experimental.pallas import tpu_sc as plsc

_NSEG = 10000
_NEDGE = 320000
_D = 128
_NC = 2
_NS = 16
_NW = _NC * _NS
_EDGES_PER_TILE = _NEDGE // _NW
_K = 80
_NITER = _EDGES_PER_TILE // _K
_NBUF = 4
_RPT = 624
_TAIL = _NSEG - _RPT * _NS

_mesh = plsc.VectorSubcoreMesh(core_axis_name="c", subcore_axis_name="s")


@functools.partial(
    pl.kernel,
    out_type=(
        jax.ShapeDtypeStruct((_NSEG, _D), jnp.float32),
        jax.ShapeDtypeStruct((_NSEG, _D), jnp.float32),
    ),
    mesh=_mesh,
    scratch_types=[
        [pltpu.VMEM((_K,), jnp.int32) for _ in range(_NBUF)],
        [pltpu.VMEM((_K, _D), jnp.float32) for _ in range(_NBUF)],
        pltpu.VMEM_SHARED((_NSEG, _D), jnp.float32),
        [pltpu.SemaphoreType.DMA for _ in range(_NBUF)],
    ],
)
def _sc_partials(feat_hbm, ids_hbm, zeros_hbm, out0, out1, idbufs, bufs, acc, sems):
    c = lax.axis_index("c")
    s = lax.axis_index("s")
    wid = s * _NC + c
    base = wid * _EDGES_PER_TILE
    r0 = s * _RPT

    for b in range(_NBUF):
        pltpu.make_async_copy(
            ids_hbm.at[pl.ds(base + b * _K, _K)], idbufs[b], sems[b]
        ).start()
        pltpu.make_async_copy(
            feat_hbm.at[pl.ds(base + b * _K, _K)], bufs[b], sems[b]
        ).start()
    pltpu.sync_copy(zeros_hbm.at[pl.ds(r0, _RPT)], acc.at[pl.ds(r0, _RPT)])

    @pl.when(s == 0)
    def _():
        pltpu.sync_copy(
            zeros_hbm.at[pl.ds(_RPT * _NS, _TAIL)], acc.at[pl.ds(_RPT * _NS, _TAIL)]
        )

    plsc.subcore_barrier()

    def run_block(g, idbuf, buf, sem):
        off = base + g * _K
        pltpu.make_async_copy(ids_hbm.at[pl.ds(off, _K)], idbuf, sem).wait()
        pltpu.make_async_copy(feat_hbm.at[pl.ds(off, _K)], buf, sem).wait()
        pltpu.sync_copy(buf, acc.at[idbuf], add=True)

        @pl.when(g + _NBUF < _NITER)
        def _():
            off2 = base + (g + _NBUF) * _K
            pltpu.make_async_copy(ids_hbm.at[pl.ds(off2, _K)], idbuf, sem).start()
            pltpu.make_async_copy(feat_hbm.at[pl.ds(off2, _K)], buf, sem).start()

    def body(g, carry):
        for b in range(_NBUF):

            @pl.when(g % _NBUF == b)
            def _(b=b):
                run_block(g, idbufs[b], bufs[b], sems[b])

        return carry

    lax.fori_loop(0, _NITER, body, 0)
    plsc.subcore_barrier()

    @pl.when(c == 0)
    def _():
        pltpu.sync_copy(acc.at[pl.ds(r0, _RPT)], out0.at[pl.ds(r0, _RPT)])

        @pl.when(s == 0)
        def _():
            pltpu.sync_copy(
                acc.at[pl.ds(_RPT * _NS, _TAIL)], out0.at[pl.ds(_RPT * _NS, _TAIL)]
            )

    @pl.when(c == 1)
    def _():
        pltpu.sync_copy(acc.at[pl.ds(r0, _RPT)], out1.at[pl.ds(r0, _RPT)])

        @pl.when(s == 0)
        def _():
            pltpu.sync_copy(
                acc.at[pl.ds(_RPT * _NS, _TAIL)], out1.at[pl.ds(_RPT * _NS, _TAIL)]
            )


def _add_body(a_ref, b_ref, o_ref):
    o_ref[...] = a_ref[...] + b_ref[...]


_combine = pl.pallas_call(
    _add_body,
    out_shape=jax.ShapeDtypeStruct((_NSEG, _D), jnp.float32),
)

_ZEROS = np.zeros((_NSEG, _D), np.float32)


def kernel(features, segment_ids):
    p0, p1 = _sc_partials(features, segment_ids, _ZEROS)
    return _combine(p0, p1)


# speedup vs baseline: 1.0995x; 1.0167x over previous
import functools

import numpy as np

import jax
import jax.numpy as jnp
from jax import lax
from jax.experimental import pallas as pl
from jax.experimental.pallas import tpu as pltpu
from jax.experimental.pallas import tpu_sc as plsc

_NSEG = 10000
_NEDGE = 320000
_D = 128
_NC = 2
_NS = 16
_NW = _NC * _NS
_EDGES_PER_TILE = _NEDGE // _NW
_K = 80
_NITER = _EDGES_PER_TILE // _K
_NBUF = 4
_RPT = 624
_TAIL = _NSEG - _RPT * _NS

_mesh = plsc.VectorSubcoreMesh(core_axis_name="c", subcore_axis_name="s")


@functools.partial(
    pl.kernel,
    out_type=(
        jax.ShapeDtypeStruct((_NSEG, _D), jnp.float32),
        jax.ShapeDtypeStruct((_NSEG, _D), jnp.float32),
    ),
    mesh=_mesh,
    scratch_types=[
        [pltpu.VMEM((_K,), jnp.int32) for _ in range(_NBUF)],
        [pltpu.VMEM((_K, _D), jnp.float32) for _ in range(_NBUF)],
        pltpu.VMEM_SHARED((_NSEG, _D), jnp.float32),
        [pltpu.SemaphoreType.DMA for _ in range(_NBUF)],
        [pltpu.SemaphoreType.DMA for _ in range(_NBUF)],
    ],
)
def _sc_partials(
    feat_hbm, ids_hbm, zeros_hbm, out0, out1, idbufs, bufs, acc, sems, ssems
):
    c = lax.axis_index("c")
    s = lax.axis_index("s")
    wid = s * _NC + c
    base = wid * _EDGES_PER_TILE
    r0 = s * _RPT

    for b in range(_NBUF):
        pltpu.make_async_copy(
            ids_hbm.at[pl.ds(base + b * _K, _K)], idbufs[b], sems[b]
        ).start()
        pltpu.make_async_copy(
            feat_hbm.at[pl.ds(base + b * _K, _K)], bufs[b], sems[b]
        ).start()
    pltpu.sync_copy(zeros_hbm.at[pl.ds(r0, _RPT)], acc.at[pl.ds(r0, _RPT)])

    @pl.when(s == 0)
    def _():
        pltpu.sync_copy(
            zeros_hbm.at[pl.ds(_RPT * _NS, _TAIL)], acc.at[pl.ds(_RPT * _NS, _TAIL)]
        )

    plsc.subcore_barrier()

    def run_block(g, b):
        off = base + g * _K
        pltpu.make_async_copy(ids_hbm.at[pl.ds(off, _K)], idbufs[b], sems[b]).wait()
        pltpu.make_async_copy(feat_hbm.at[pl.ds(off, _K)], bufs[b], sems[b]).wait()
        pltpu.async_copy(bufs[b], acc.at[idbufs[b]], ssems[b], add=True)

        @pl.when(g >= 1)
        def _():
            bp = (b - 1) % _NBUF
            pltpu.make_async_copy(bufs[bp], acc.at[idbufs[bp]], ssems[bp]).wait()

            @pl.when(g - 1 + _NBUF < _NITER)
            def _():
                off2 = base + (g - 1 + _NBUF) * _K
                pltpu.make_async_copy(
                    ids_hbm.at[pl.ds(off2, _K)], idbufs[bp], sems[bp]
                ).start()
                pltpu.make_async_copy(
                    feat_hbm.at[pl.ds(off2, _K)], bufs[bp], sems[bp]
                ).start()

    def body(g, carry):
        for b in range(_NBUF):

            @pl.when(g % _NBUF == b)
            def _(b=b):
                run_block(g, b)

        return carry

    lax.fori_loop(0, _NITER, body, 0)
    bl = (_NITER - 1) % _NBUF
    pltpu.make_async_copy(bufs[bl], acc.at[idbufs[bl]], ssems[bl]).wait()
    plsc.subcore_barrier()

    @pl.when(c == 0)
    def _():
        pltpu.sync_copy(acc.at[pl.ds(r0, _RPT)], out0.at[pl.ds(r0, _RPT)])

        @pl.when(s == 0)
        def _():
            pltpu.sync_copy(
                acc.at[pl.ds(_RPT * _NS, _TAIL)], out0.at[pl.ds(_RPT * _NS, _TAIL)]
            )

    @pl.when(c == 1)
    def _():
        pltpu.sync_copy(acc.at[pl.ds(r0, _RPT)], out1.at[pl.ds(r0, _RPT)])

        @pl.when(s == 0)
        def _():
            pltpu.sync_copy(
                acc.at[pl.ds(_RPT * _NS, _TAIL)], out1.at[pl.ds(_RPT * _NS, _TAIL)]
            )


def _add_body(a_ref, b_ref, o_ref):
    o_ref[...] = a_ref[...] + b_ref[...]


_combine = pl.pallas_call(
    _add_body,
    out_shape=jax.ShapeDtypeStruct((_NSEG, _D), jnp.float32),
)

_ZEROS = np.zeros((_NSEG, _D), np.float32)


def kernel(features, segment_ids):
    p0, p1 = _sc_partials(features, segment_ids, _ZEROS)
    return _combine(p0, p1)
